# baseline (speedup 1.0000x reference)
import functools
import math

import jax
import jax.numpy as jnp
from jax import lax
from jax.experimental import pallas as pl
from jax.experimental.pallas import tpu as pltpu
from jax.experimental.pallas import tpu_sc as plsc

N = 10000
E = 160000
D = 256
H = 4
C = 64
HC = 256
OUT = 256
EPS = 1e-5

NS = 16
LANES = 16
EPSUB = E // NS
M = 80
NM = EPSUB // M
G = M // LANES
NP = 10240
HH = 128


def _proj_body(x_ref, w_ref, b_ref, kv_ref, q_ref, s_ref):
    acc = jnp.dot(x_ref[...], w_ref[...], preferred_element_type=jnp.float32)
    for c in range(2):
        kv_ref[c, :, 0:128] = acc[:, 128 * c:128 * c + 128] + b_ref[c][None, :]
        kv_ref[c, :, 128:256] = \
            acc[:, 512 + 128 * c:512 + 128 * c + 128] + b_ref[4 + c][None, :]
        q_ref[c] = acc[:, 256 + 128 * c:256 + 128 * c + 128] \
            + b_ref[2 + c][None, :]
        s_ref[c] = acc[:, 768 + 128 * c:768 + 128 * c + 128] \
            + b_ref[6 + c][None, :]


def _proj(x, wcat, bcat):
    return pl.pallas_call(
        _proj_body,
        out_shape=(
            jax.ShapeDtypeStruct((2, N, 256), jnp.float32),
            jax.ShapeDtypeStruct((2, N, 128), jnp.float32),
            jax.ShapeDtypeStruct((2, N, 128), jnp.float32),
        ),
    )(x, wcat, bcat)


def _bnrelu_body(agg_ref, s_ref, g_ref, b_ref, out_ref):
    for c in range(2):
        t = agg_ref[c] + s_ref[c]
        m = jnp.mean(t, axis=0)
        t0 = t - m[None, :]
        v = jnp.mean(t0 * t0, axis=0)
        y = g_ref[c][None, :] * t0 * lax.rsqrt(v + EPS) + b_ref[c][None, :]
        out_ref[:, 128 * c:128 * (c + 1)] = jnp.maximum(y, 0.0)


def _bnrelu(agg, s, g, b):
    return pl.pallas_call(
        _bnrelu_body,
        out_shape=jax.ShapeDtypeStruct((N, HC), jnp.float32),
    )(agg, s, g.reshape(2, 128), b.reshape(2, 128))


def _final_body(h_ref, w_ref, b_ref, g_ref, bn_ref, out_ref):
    t = jnp.dot(h_ref[...], w_ref[...], preferred_element_type=jnp.float32)
    t = t + b_ref[...].reshape(1, OUT)
    m = jnp.mean(t, axis=0)
    t0 = t - m[None, :]
    v = jnp.mean(t0 * t0, axis=0)
    y = g_ref[...].reshape(1, OUT) * t0 * lax.rsqrt(v + EPS) \
        + bn_ref[...].reshape(1, OUT)
    out_ref[...] = jnp.maximum(y, 0.0)


def _final(h, w, b, g, bn):
    return pl.pallas_call(
        _final_body,
        out_shape=jax.ShapeDtypeStruct((N, OUT), jnp.float32),
    )(h, w, b.reshape(2, 128), g.reshape(2, 128), bn.reshape(2, 128))


def _attn_body(kv2, q2, srch, dsth, out,
               sidx, didx, gidx, kvbuf, qbuf, mbuf,
               exb0, exb1, tr0, tr1, zrow, dfinal, agg, sem):
    c = lax.axis_index("c")
    s = lax.axis_index("s")
    cn = c * N
    cnv = jnp.full((LANES,), cn, jnp.int32)
    npv = jnp.full((LANES,), NP, jnp.int32)
    iot = lax.iota(jnp.int32, LANES)
    ib16 = iot * LANES
    zf = jnp.zeros((LANES,), jnp.float32)

    def zrow_body(i, _):
        zrow[pl.ds(i * LANES, LANES)] = zf
        return 0
    lax.fori_loop(0, 1280 // LANES, zrow_body, 0)

    def zbuf_body(r, _):
        for j in range(8):
            mbuf[r, pl.ds(j * LANES, LANES)] = zf
        return 0
    lax.fori_loop(0, M, zbuf_body, 0)

    for t in range(8):
        ch = s + NS * t

        @pl.when(ch < NM)
        def _():
            pltpu.sync_copy(mbuf, agg.at[pl.ds(ch * M, M)])
    pltpu.sync_copy(zrow, dfinal.at[pl.ds(s * 1280, 1280)])
    plsc.subcore_barrier()

    ebase = s * EPSUB

    def fill_gidx(src_ref, addv):
        def body(g, _):
            gidx[pl.ds(g * LANES, LANES)] = \
                src_ref[pl.ds(g * LANES, LANES)] + addv
            return 0
        lax.fori_loop(0, G, body, 0)

    def sweep(i, _):
        eb = ebase + i * M
        pltpu.sync_copy(srch.at[pl.ds(eb, M)], sidx)
        pltpu.sync_copy(dsth.at[pl.ds(eb, M)], didx)
        fill_gidx(sidx, cnv)
        pltpu.async_copy(kv2.at[gidx], kvbuf, sem).wait()
        fill_gidx(didx, cnv)
        pltpu.async_copy(q2.at[gidx], qbuf, sem).wait()

        def group(g, _):
            def edge(e16, _):
                e = g * LANES + e16
                acc0 = zf
                acc1 = zf
                for j in range(8):
                    kvv = kvbuf[e, pl.ds(j * LANES, LANES)]
                    qv = qbuf[e, pl.ds(j * LANES, LANES)]
                    if j < 4:
                        acc0 = acc0 + kvv * qv
                    else:
                        acc1 = acc1 + kvv * qv
                tr0[pl.ds(e16 * LANES, LANES)] = acc0
                tr1[pl.ds(e16 * LANES, LANES)] = acc1
                return 0
            lax.fori_loop(0, LANES, edge, 0)

            s0 = zf
            s1 = zf
            for j in range(LANES):
                s0 = s0 + plsc.load_gather(tr0, [ib16 + j])
                s1 = s1 + plsc.load_gather(tr1, [ib16 + j])
            exb0[pl.ds(g * LANES, LANES)] = jnp.exp(s0 * 0.125)
            exb1[pl.ds(g * LANES, LANES)] = jnp.exp(s1 * 0.125)
            return 0
        lax.fori_loop(0, G, group, 0)

        fill_gidx(didx, npv)
        pltpu.sync_copy(exb0, dfinal.at[didx], add=True)
        pltpu.sync_copy(exb1, dfinal.at[gidx], add=True)

        def edge(e, _):
            ev = jnp.full((LANES,), e, jnp.int32)
            b0 = plsc.load_gather(exb0, [ev])
            b1 = plsc.load_gather(exb1, [ev])
            for j in range(8):
                bb = b0 if j < 4 else b1
                mbuf[e, pl.ds(j * LANES, LANES)] = \
                    kvbuf[e, pl.ds(128 + j * LANES, LANES)] * bb
            return 0
        lax.fori_loop(0, M, edge, 0)

        pltpu.sync_copy(mbuf, agg.at[didx], add=True)

        return 0
    lax.fori_loop(0, NM, sweep, 0)

    plsc.subcore_barrier()

    for t in range(8):
        ch = s + NS * t

        @pl.when(ch < NM)
        def _():
            base = ch * M
            pltpu.sync_copy(agg.at[pl.ds(base, M)], mbuf)
            pltpu.sync_copy(dfinal.at[pl.ds(base, M)], exb0)
            pltpu.sync_copy(dfinal.at[pl.ds(NP + base, M)], exb1)

            def recip(g, _):
                exb0[pl.ds(g * LANES, LANES)] = \
                    1.0 / (exb0[pl.ds(g * LANES, LANES)] + 1e-16)
                exb1[pl.ds(g * LANES, LANES)] = \
                    1.0 / (exb1[pl.ds(g * LANES, LANES)] + 1e-16)
                return 0
            lax.fori_loop(0, G, recip, 0)

            def row(r, _):
                rv = jnp.full((LANES,), r, jnp.int32)
                b0 = plsc.load_gather(exb0, [rv])
                b1 = plsc.load_gather(exb1, [rv])
                for j in range(8):
                    bb = b0 if j < 4 else b1
                    mbuf[r, pl.ds(j * LANES, LANES)] = \
                        mbuf[r, pl.ds(j * LANES, LANES)] * bb
                return 0
            lax.fori_loop(0, M, row, 0)

            pltpu.sync_copy(mbuf, out.at[c].at[pl.ds(base, M)])


@functools.partial(
    pl.kernel,
    out_type=jax.ShapeDtypeStruct((2, N, HH), jnp.float32),
    mesh=plsc.VectorSubcoreMesh(core_axis_name="c", subcore_axis_name="s"),
    compiler_params=pltpu.CompilerParams(needs_layout_passes=False),
    scratch_types=[
        pltpu.VMEM((M,), jnp.int32),
        pltpu.VMEM((M,), jnp.int32),
        pltpu.VMEM((M,), jnp.int32),
        pltpu.VMEM((M, 256), jnp.float32),
        pltpu.VMEM((M, HH), jnp.float32),
        pltpu.VMEM((M, HH), jnp.float32),
        pltpu.VMEM((M,), jnp.float32),
        pltpu.VMEM((M,), jnp.float32),
        pltpu.VMEM((LANES * LANES,), jnp.float32),
        pltpu.VMEM((LANES * LANES,), jnp.float32),
        pltpu.VMEM((1280,), jnp.float32),
        pltpu.VMEM_SHARED((2 * NP,), jnp.float32),
        pltpu.VMEM_SHARED((N, HH), jnp.float32),
        pltpu.SemaphoreType.DMA,
    ],
)
def _attn(kv2, q2, srch, dsth, out, *scratch):
    _attn_body(kv2, q2, srch, dsth, out, *scratch)


def _tconv(x, src, dst, wcat, bcat):
    kv, q, s2 = _proj(x, wcat, bcat)
    kv2 = kv.reshape(2 * N, 256)
    q2 = q.reshape(2 * N, HH)
    agg = _attn(kv2, q2, src, dst)
    return agg, s2


def kernel(x, edge_index, l0_Wk, l0_bk, l0_Wq, l0_bq, l0_Wv, l0_bv, l0_Ws,
           l0_bs, l1_Wk, l1_bk, l1_Wq, l1_bq, l1_Wv, l1_bv, l1_Ws, l1_bs,
           bn0_g, bn0_b, bn1_g, bn1_b, bn2_g, bn2_b, Wout, bout):
    src = edge_index[0]
    dst = edge_index[1]
    w0 = jnp.concatenate([l0_Wk, l0_Wq, l0_Wv, l0_Ws], axis=1)
    b0 = jnp.concatenate([l0_bk, l0_bq, l0_bv, l0_bs]).reshape(8, 128)
    w1 = jnp.concatenate([l1_Wk, l1_Wq, l1_Wv, l1_Ws], axis=1)
    b1 = jnp.concatenate([l1_bk, l1_bq, l1_bv, l1_bs]).reshape(8, 128)

    agg, s2 = _tconv(x, src, dst, w0, b0)
    h = _bnrelu(agg, s2, bn0_g, bn0_b)
    agg, s2 = _tconv(h, src, dst, w1, b1)
    h = _bnrelu(agg, s2, bn1_g, bn1_b)
    return _final(h, Wout, bout, bn2_g, bn2_b)


# speedup vs baseline: 1.0757x; 1.0757x over previous
import functools
import math

import jax
import jax.numpy as jnp
from jax import lax
from jax.experimental import pallas as pl
from jax.experimental.pallas import tpu as pltpu
from jax.experimental.pallas import tpu_sc as plsc

N = 10000
E = 160000
D = 256
H = 4
C = 64
HC = 256
OUT = 256
EPS = 1e-5

NS = 16
LANES = 16
EPSUB = E // NS
M = 80
NM = EPSUB // M
G = M // LANES
NP = 10240
HH = 128


def _proj_body(x_ref, w_ref, b_ref, kv_ref, q_ref, s_ref):
    acc = jnp.dot(x_ref[...], w_ref[...], preferred_element_type=jnp.float32)
    for c in range(2):
        kv_ref[c, :, 0:128] = acc[:, 128 * c:128 * c + 128] + b_ref[c][None, :]
        kv_ref[c, :, 128:256] = \
            acc[:, 512 + 128 * c:512 + 128 * c + 128] + b_ref[4 + c][None, :]
        q_ref[c] = acc[:, 256 + 128 * c:256 + 128 * c + 128] \
            + b_ref[2 + c][None, :]
        s_ref[c] = acc[:, 768 + 128 * c:768 + 128 * c + 128] \
            + b_ref[6 + c][None, :]


def _proj(x, wcat, bcat):
    return pl.pallas_call(
        _proj_body,
        out_shape=(
            jax.ShapeDtypeStruct((2, N, 256), jnp.float32),
            jax.ShapeDtypeStruct((2, N, 128), jnp.float32),
            jax.ShapeDtypeStruct((2, N, 128), jnp.float32),
        ),
    )(x, wcat, bcat)


def _bnrelu_body(agg_ref, s_ref, g_ref, b_ref, out_ref):
    for c in range(2):
        t = agg_ref[c] + s_ref[c]
        m = jnp.mean(t, axis=0)
        t0 = t - m[None, :]
        v = jnp.mean(t0 * t0, axis=0)
        y = g_ref[c][None, :] * t0 * lax.rsqrt(v + EPS) + b_ref[c][None, :]
        out_ref[:, 128 * c:128 * (c + 1)] = jnp.maximum(y, 0.0)


def _bnrelu(agg, s, g, b):
    return pl.pallas_call(
        _bnrelu_body,
        out_shape=jax.ShapeDtypeStruct((N, HC), jnp.float32),
    )(agg, s, g.reshape(2, 128), b.reshape(2, 128))


def _final_body(h_ref, w_ref, b_ref, g_ref, bn_ref, out_ref):
    t = jnp.dot(h_ref[...], w_ref[...], preferred_element_type=jnp.float32)
    t = t + b_ref[...].reshape(1, OUT)
    m = jnp.mean(t, axis=0)
    t0 = t - m[None, :]
    v = jnp.mean(t0 * t0, axis=0)
    y = g_ref[...].reshape(1, OUT) * t0 * lax.rsqrt(v + EPS) \
        + bn_ref[...].reshape(1, OUT)
    out_ref[...] = jnp.maximum(y, 0.0)


def _final(h, w, b, g, bn):
    return pl.pallas_call(
        _final_body,
        out_shape=jax.ShapeDtypeStruct((N, OUT), jnp.float32),
    )(h, w, b.reshape(2, 128), g.reshape(2, 128), bn.reshape(2, 128))


def _attn_body(kv2, q2, srch, dsth, out,
               srcC, dstC, gs, gd, sdst, sdst1,
               kvb, qb, mbuf,
               exb0, exb1, tr0, tr1, zrow, dfinal, agg, sem):
    c = lax.axis_index("c")
    s = lax.axis_index("s")
    cn = c * N
    cnv = jnp.full((LANES,), cn, jnp.int32)
    npv = jnp.full((LANES,), NP, jnp.int32)
    ziv = jnp.zeros((LANES,), jnp.int32)
    iot = lax.iota(jnp.int32, LANES)
    ib16 = iot * LANES
    zf = jnp.zeros((LANES,), jnp.float32)

    def zrow_body(i, _):
        zrow[pl.ds(i * LANES, LANES)] = zf
        return 0
    lax.fori_loop(0, 1280 // LANES, zrow_body, 0)

    def zbuf_body(r, _):
        for j in range(8):
            mbuf[r, pl.ds(j * LANES, LANES)] = zf
        return 0
    lax.fori_loop(0, M, zbuf_body, 0)

    for t in range(8):
        ch = s + NS * t

        @pl.when(ch < NM)
        def _():
            pltpu.sync_copy(mbuf, agg.at[pl.ds(ch * M, M)])
    pltpu.sync_copy(zrow, dfinal.at[pl.ds(s * 1280, 1280)])
    plsc.subcore_barrier()

    ebase = s * EPSUB

    def build(idx_ref, from_ref, addv):
        def body(g, _):
            idx_ref[pl.ds(g * LANES, LANES)] = \
                from_ref[pl.ds(g * LANES, LANES)] + addv
            return 0
        lax.fori_loop(0, G, body, 0)

    def compute():
        def group(g, _):
            def edge(e16, _):
                e = g * LANES + e16
                acc0 = zf
                acc1 = zf
                for j in range(8):
                    kvv = kvb[e, pl.ds(j * LANES, LANES)]
                    qv = qb[e, pl.ds(j * LANES, LANES)]
                    if j < 4:
                        acc0 = acc0 + kvv * qv
                    else:
                        acc1 = acc1 + kvv * qv
                tr0[pl.ds(e16 * LANES, LANES)] = acc0
                tr1[pl.ds(e16 * LANES, LANES)] = acc1
                return 0
            lax.fori_loop(0, LANES, edge, 0)

            s0 = zf
            s1 = zf
            for j in range(LANES):
                s0 = s0 + plsc.load_gather(tr0, [ib16 + j])
                s1 = s1 + plsc.load_gather(tr1, [ib16 + j])
            exb0[pl.ds(g * LANES, LANES)] = jnp.exp(s0 * 0.125)
            exb1[pl.ds(g * LANES, LANES)] = jnp.exp(s1 * 0.125)
            return 0
        lax.fori_loop(0, G, group, 0)

        def edge(e, _):
            ev = jnp.full((LANES,), e, jnp.int32)
            b0 = plsc.load_gather(exb0, [ev])
            b1 = plsc.load_gather(exb1, [ev])
            for j in range(8):
                bb = b0 if j < 4 else b1
                mbuf[e, pl.ds(j * LANES, LANES)] = \
                    kvb[e, pl.ds(128 + j * LANES, LANES)] * bb
            return 0
        lax.fori_loop(0, M, edge, 0)

        pltpu.sync_copy(exb0, dfinal.at[sdst], add=True)
        pltpu.sync_copy(exb1, dfinal.at[sdst1], add=True)
        pltpu.sync_copy(mbuf, agg.at[sdst], add=True)

    def chunk(i, _):
        pltpu.sync_copy(srch.at[pl.ds(ebase + i * M, M)], srcC)
        pltpu.sync_copy(dsth.at[pl.ds(ebase + i * M, M)], dstC)
        build(gs, srcC, cnv)
        build(gd, dstC, cnv)
        pltpu.async_copy(kv2.at[gs], kvb, sem)
        pltpu.async_copy(q2.at[gd], qb, sem)
        build(sdst, dstC, ziv)
        build(sdst1, dstC, npv)
        pltpu.make_async_copy(kv2.at[gs], kvb, sem).wait()
        pltpu.make_async_copy(q2.at[gd], qb, sem).wait()
        compute()
        return 0
    lax.fori_loop(0, NM, chunk, 0)

    plsc.subcore_barrier()

    for t in range(8):
        ch = s + NS * t

        @pl.when(ch < NM)
        def _():
            base = ch * M
            pltpu.sync_copy(agg.at[pl.ds(base, M)], mbuf)
            pltpu.sync_copy(dfinal.at[pl.ds(base, M)], exb0)
            pltpu.sync_copy(dfinal.at[pl.ds(NP + base, M)], exb1)

            def recip(g, _):
                exb0[pl.ds(g * LANES, LANES)] = \
                    1.0 / (exb0[pl.ds(g * LANES, LANES)] + 1e-16)
                exb1[pl.ds(g * LANES, LANES)] = \
                    1.0 / (exb1[pl.ds(g * LANES, LANES)] + 1e-16)
                return 0
            lax.fori_loop(0, G, recip, 0)

            def row(r, _):
                rv = jnp.full((LANES,), r, jnp.int32)
                b0 = plsc.load_gather(exb0, [rv])
                b1 = plsc.load_gather(exb1, [rv])
                for j in range(8):
                    bb = b0 if j < 4 else b1
                    mbuf[r, pl.ds(j * LANES, LANES)] = \
                        mbuf[r, pl.ds(j * LANES, LANES)] * bb
                return 0
            lax.fori_loop(0, M, row, 0)

            pltpu.sync_copy(mbuf, out.at[c].at[pl.ds(base, M)])


@functools.partial(
    pl.kernel,
    out_type=jax.ShapeDtypeStruct((2, N, HH), jnp.float32),
    mesh=plsc.VectorSubcoreMesh(core_axis_name="c", subcore_axis_name="s"),
    compiler_params=pltpu.CompilerParams(needs_layout_passes=False),
    scratch_types=[
        pltpu.VMEM((M,), jnp.int32),
        pltpu.VMEM((M,), jnp.int32),
        pltpu.VMEM((M,), jnp.int32),
        pltpu.VMEM((M,), jnp.int32),
        pltpu.VMEM((M,), jnp.int32),
        pltpu.VMEM((M,), jnp.int32),
        pltpu.VMEM((M, 256), jnp.float32),
        pltpu.VMEM((M, HH), jnp.float32),
        pltpu.VMEM((M, HH), jnp.float32),
        pltpu.VMEM((M,), jnp.float32),
        pltpu.VMEM((M,), jnp.float32),
        pltpu.VMEM((LANES * LANES,), jnp.float32),
        pltpu.VMEM((LANES * LANES,), jnp.float32),
        pltpu.VMEM((1280,), jnp.float32),
        pltpu.VMEM_SHARED((2 * NP,), jnp.float32),
        pltpu.VMEM_SHARED((N, HH), jnp.float32),
        pltpu.SemaphoreType.DMA,
    ],
)
def _attn(kv2, q2, srch, dsth, out, *scratch):
    _attn_body(kv2, q2, srch, dsth, out, *scratch)


def _tconv(x, src, dst, wcat, bcat):
    kv, q, s2 = _proj(x, wcat, bcat)
    kv2 = kv.reshape(2 * N, 256)
    q2 = q.reshape(2 * N, HH)
    agg = _attn(kv2, q2, src, dst)
    return agg, s2


def kernel(x, edge_index, l0_Wk, l0_bk, l0_Wq, l0_bq, l0_Wv, l0_bv, l0_Ws,
           l0_bs, l1_Wk, l1_bk, l1_Wq, l1_bq, l1_Wv, l1_bv, l1_Ws, l1_bs,
           bn0_g, bn0_b, bn1_g, bn1_b, bn2_g, bn2_b, Wout, bout):
    src = edge_index[0]
    dst = edge_index[1]
    w0 = jnp.concatenate([l0_Wk, l0_Wq, l0_Wv, l0_Ws], axis=1)
    b0 = jnp.concatenate([l0_bk, l0_bq, l0_bv, l0_bs]).reshape(8, 128)
    w1 = jnp.concatenate([l1_Wk, l1_Wq, l1_Wv, l1_Ws], axis=1)
    b1 = jnp.concatenate([l1_bk, l1_bq, l1_bv, l1_bs]).reshape(8, 128)

    agg, s2 = _tconv(x, src, dst, w0, b0)
    h = _bnrelu(agg, s2, bn0_g, bn0_b)
    agg, s2 = _tconv(h, src, dst, w1, b1)
    h = _bnrelu(agg, s2, bn1_g, bn1_b)
    return _final(h, Wout, bout, bn2_g, bn2_b)


# speedup vs baseline: 1.3540x; 1.2586x over previous
import functools
import math

import jax
import jax.numpy as jnp
from jax import lax
from jax.experimental import pallas as pl
from jax.experimental.pallas import tpu as pltpu
from jax.experimental.pallas import tpu_sc as plsc

N = 10000
E = 160000
D = 256
H = 4
C = 64
HC = 256
OUT = 256
EPS = 1e-5

NS = 16
LANES = 16
EPSUB = E // NS
M = 80
NM = EPSUB // M
G = M // LANES
NP = 10240
HH = 128


def _proj_body(x_ref, w_ref, b_ref, k_ref, v_ref, q_ref, s_ref):
    acc = jnp.dot(x_ref[...], w_ref[...], preferred_element_type=jnp.float32)
    for c in range(2):
        k_ref[c] = acc[:, 128 * c:128 * c + 128] + b_ref[c][None, :]
        v_ref[c] = acc[:, 512 + 128 * c:512 + 128 * c + 128] \
            + b_ref[4 + c][None, :]
        q_ref[c] = acc[:, 256 + 128 * c:256 + 128 * c + 128] \
            + b_ref[2 + c][None, :]
        s_ref[c] = acc[:, 768 + 128 * c:768 + 128 * c + 128] \
            + b_ref[6 + c][None, :]


def _proj(x, wcat, bcat):
    return pl.pallas_call(
        _proj_body,
        out_shape=(
            jax.ShapeDtypeStruct((2, N, 128), jnp.float32),
            jax.ShapeDtypeStruct((2, N, 128), jnp.float32),
            jax.ShapeDtypeStruct((2, N, 128), jnp.float32),
            jax.ShapeDtypeStruct((2, N, 128), jnp.float32),
        ),
    )(x, wcat, bcat)


def _bnrelu_body(agg_ref, s_ref, g_ref, b_ref, out_ref):
    for c in range(2):
        t = agg_ref[c] + s_ref[c]
        m = jnp.mean(t, axis=0)
        t0 = t - m[None, :]
        v = jnp.mean(t0 * t0, axis=0)
        y = g_ref[c][None, :] * t0 * lax.rsqrt(v + EPS) + b_ref[c][None, :]
        out_ref[:, 128 * c:128 * (c + 1)] = jnp.maximum(y, 0.0)


def _bnrelu(agg, s, g, b):
    return pl.pallas_call(
        _bnrelu_body,
        out_shape=jax.ShapeDtypeStruct((N, HC), jnp.float32),
    )(agg, s, g.reshape(2, 128), b.reshape(2, 128))


def _final_body(h_ref, w_ref, b_ref, g_ref, bn_ref, out_ref):
    t = jnp.dot(h_ref[...], w_ref[...], preferred_element_type=jnp.float32)
    t = t + b_ref[...].reshape(1, OUT)
    m = jnp.mean(t, axis=0)
    t0 = t - m[None, :]
    v = jnp.mean(t0 * t0, axis=0)
    y = g_ref[...].reshape(1, OUT) * t0 * lax.rsqrt(v + EPS) \
        + bn_ref[...].reshape(1, OUT)
    out_ref[...] = jnp.maximum(y, 0.0)


def _final(h, w, b, g, bn):
    return pl.pallas_call(
        _final_body,
        out_shape=jax.ShapeDtypeStruct((N, OUT), jnp.float32),
    )(h, w, b.reshape(2, 128), g.reshape(2, 128), bn.reshape(2, 128))


def _attn_body(k2, v2, q2, srch, dsth, out,
               srcC, dstC, gs, gd, gv, sdst, sdst1,
               kb, vb, qb, mbuf,
               exb0, exb1, tr0, tr1, zrow, dfinal, agg, semA, semB):
    c = lax.axis_index("c")
    s = lax.axis_index("s")
    cn = c * N
    cnv = jnp.full((LANES,), cn, jnp.int32)
    npv = jnp.full((LANES,), NP, jnp.int32)
    ziv = jnp.zeros((LANES,), jnp.int32)
    iot = lax.iota(jnp.int32, LANES)
    ib16 = iot * LANES
    zf = jnp.zeros((LANES,), jnp.float32)

    def zrow_body(i, _):
        zrow[pl.ds(i * LANES, LANES)] = zf
        return 0
    lax.fori_loop(0, 1280 // LANES, zrow_body, 0)

    def zbuf_body(r, _):
        for j in range(8):
            mbuf[r, pl.ds(j * LANES, LANES)] = zf
        return 0
    lax.fori_loop(0, M, zbuf_body, 0)

    for t in range(8):
        ch = s + NS * t

        @pl.when(ch < NM)
        def _():
            pltpu.sync_copy(mbuf, agg.at[pl.ds(ch * M, M)])
    pltpu.sync_copy(zrow, dfinal.at[pl.ds(s * 1280, 1280)])
    plsc.subcore_barrier()

    ebase = s * EPSUB

    def build(idx_ref, from_ref, addv):
        def body(g, _):
            idx_ref[pl.ds(g * LANES, LANES)] = \
                from_ref[pl.ds(g * LANES, LANES)] + addv
            return 0
        lax.fori_loop(0, G, body, 0)

    def compute_dots():
        def group(g, _):
            def edge(e16, _):
                e = g * LANES + e16
                acc0 = zf
                acc1 = zf
                for j in range(8):
                    kvv = kb[e, pl.ds(j * LANES, LANES)]
                    qv = qb[e, pl.ds(j * LANES, LANES)]
                    if j < 4:
                        acc0 = acc0 + kvv * qv
                    else:
                        acc1 = acc1 + kvv * qv
                tr0[pl.ds(e16 * LANES, LANES)] = acc0
                tr1[pl.ds(e16 * LANES, LANES)] = acc1
                return 0
            lax.fori_loop(0, LANES, edge, 0)

            s0 = zf
            s1 = zf
            for j in range(LANES):
                s0 = s0 + plsc.load_gather(tr0, [ib16 + j])
                s1 = s1 + plsc.load_gather(tr1, [ib16 + j])
            exb0[pl.ds(g * LANES, LANES)] = jnp.exp(s0 * 0.125)
            exb1[pl.ds(g * LANES, LANES)] = jnp.exp(s1 * 0.125)
            return 0
        lax.fori_loop(0, G, group, 0)

    def scale_and_scatter():
        def edge(e, _):
            ev = jnp.full((LANES,), e, jnp.int32)
            b0 = plsc.load_gather(exb0, [ev])
            b1 = plsc.load_gather(exb1, [ev])
            for j in range(8):
                bb = b0 if j < 4 else b1
                mbuf[e, pl.ds(j * LANES, LANES)] = \
                    vb[e, pl.ds(j * LANES, LANES)] * bb
            return 0
        lax.fori_loop(0, M, edge, 0)

        pltpu.sync_copy(exb0, dfinal.at[sdst], add=True)
        pltpu.sync_copy(exb1, dfinal.at[sdst1], add=True)
        pltpu.sync_copy(mbuf, agg.at[sdst], add=True)

    def load_idx(i):
        pltpu.sync_copy(srch.at[pl.ds(ebase + i * M, M)], srcC)
        pltpu.sync_copy(dsth.at[pl.ds(ebase + i * M, M)], dstC)

    load_idx(0)
    build(gs, srcC, cnv)
    build(gd, dstC, cnv)
    pltpu.async_copy(k2.at[gs], kb, semA)
    pltpu.async_copy(q2.at[gd], qb, semA)

    def chunk(i, _):
        build(sdst, dstC, ziv)
        build(sdst1, dstC, npv)
        build(gv, srcC, cnv)
        pltpu.make_async_copy(k2.at[gs], kb, semA).wait()
        pltpu.make_async_copy(q2.at[gd], qb, semA).wait()
        pltpu.async_copy(v2.at[gv], vb, semB)
        compute_dots()

        @pl.when(i + 1 < NM)
        def _():
            load_idx(i + 1)
            build(gs, srcC, cnv)
            build(gd, dstC, cnv)
            pltpu.async_copy(k2.at[gs], kb, semA)
            pltpu.async_copy(q2.at[gd], qb, semA)

        pltpu.make_async_copy(v2.at[gv], vb, semB).wait()
        scale_and_scatter()
        return 0
    lax.fori_loop(0, NM, chunk, 0)

    plsc.subcore_barrier()

    for t in range(8):
        ch = s + NS * t

        @pl.when(ch < NM)
        def _():
            base = ch * M
            pltpu.sync_copy(agg.at[pl.ds(base, M)], mbuf)
            pltpu.sync_copy(dfinal.at[pl.ds(base, M)], exb0)
            pltpu.sync_copy(dfinal.at[pl.ds(NP + base, M)], exb1)

            def recip(g, _):
                exb0[pl.ds(g * LANES, LANES)] = \
                    1.0 / (exb0[pl.ds(g * LANES, LANES)] + 1e-16)
                exb1[pl.ds(g * LANES, LANES)] = \
                    1.0 / (exb1[pl.ds(g * LANES, LANES)] + 1e-16)
                return 0
            lax.fori_loop(0, G, recip, 0)

            def row(r, _):
                rv = jnp.full((LANES,), r, jnp.int32)
                b0 = plsc.load_gather(exb0, [rv])
                b1 = plsc.load_gather(exb1, [rv])
                for j in range(8):
                    bb = b0 if j < 4 else b1
                    mbuf[r, pl.ds(j * LANES, LANES)] = \
                        mbuf[r, pl.ds(j * LANES, LANES)] * bb
                return 0
            lax.fori_loop(0, M, row, 0)

            pltpu.sync_copy(mbuf, out.at[c].at[pl.ds(base, M)])


@functools.partial(
    pl.kernel,
    out_type=jax.ShapeDtypeStruct((2, N, HH), jnp.float32),
    mesh=plsc.VectorSubcoreMesh(core_axis_name="c", subcore_axis_name="s"),
    compiler_params=pltpu.CompilerParams(needs_layout_passes=False),
    scratch_types=[
        pltpu.VMEM((M,), jnp.int32),
        pltpu.VMEM((M,), jnp.int32),
        pltpu.VMEM((M,), jnp.int32),
        pltpu.VMEM((M,), jnp.int32),
        pltpu.VMEM((M,), jnp.int32),
        pltpu.VMEM((M,), jnp.int32),
        pltpu.VMEM((M,), jnp.int32),
        pltpu.VMEM((M, HH), jnp.float32),
        pltpu.VMEM((M, HH), jnp.float32),
        pltpu.VMEM((M, HH), jnp.float32),
        pltpu.VMEM((M, HH), jnp.float32),
        pltpu.VMEM((M,), jnp.float32),
        pltpu.VMEM((M,), jnp.float32),
        pltpu.VMEM((LANES * LANES,), jnp.float32),
        pltpu.VMEM((LANES * LANES,), jnp.float32),
        pltpu.VMEM((1280,), jnp.float32),
        pltpu.VMEM_SHARED((2 * NP,), jnp.float32),
        pltpu.VMEM_SHARED((N, HH), jnp.float32),
        pltpu.SemaphoreType.DMA,
        pltpu.SemaphoreType.DMA,
    ],
)
def _attn(k2, v2, q2, srch, dsth, out, *scratch):
    _attn_body(k2, v2, q2, srch, dsth, out, *scratch)


def _tconv(x, src, dst, wcat, bcat):
    k, v, q, s2 = _proj(x, wcat, bcat)
    k2 = k.reshape(2 * N, HH)
    v2 = v.reshape(2 * N, HH)
    q2 = q.reshape(2 * N, HH)
    agg = _attn(k2, v2, q2, src, dst)
    return agg, s2


def kernel(x, edge_index, l0_Wk, l0_bk, l0_Wq, l0_bq, l0_Wv, l0_bv, l0_Ws,
           l0_bs, l1_Wk, l1_bk, l1_Wq, l1_bq, l1_Wv, l1_bv, l1_Ws, l1_bs,
           bn0_g, bn0_b, bn1_g, bn1_b, bn2_g, bn2_b, Wout, bout):
    src = edge_index[0]
    dst = edge_index[1]
    w0 = jnp.concatenate([l0_Wk, l0_Wq, l0_Wv, l0_Ws], axis=1)
    b0 = jnp.concatenate([l0_bk, l0_bq, l0_bv, l0_bs]).reshape(8, 128)
    w1 = jnp.concatenate([l1_Wk, l1_Wq, l1_Wv, l1_Ws], axis=1)
    b1 = jnp.concatenate([l1_bk, l1_bq, l1_bv, l1_bs]).reshape(8, 128)

    agg, s2 = _tconv(x, src, dst, w0, b0)
    h = _bnrelu(agg, s2, bn0_g, bn0_b)
    agg, s2 = _tconv(h, src, dst, w1, b1)
    h = _bnrelu(agg, s2, bn1_g, bn1_b)
    return _final(h, Wout, bout, bn2_g, bn2_b)


# speedup vs baseline: 1.5477x; 1.1431x over previous
import functools
import math

import jax
import jax.numpy as jnp
from jax import lax
from jax.experimental import pallas as pl
from jax.experimental.pallas import tpu as pltpu
from jax.experimental.pallas import tpu_sc as plsc

N = 10000
E = 160000
D = 256
H = 4
C = 64
HC = 256
OUT = 256
EPS = 1e-5

NS = 16
LANES = 16
EPSUB = E // NS
M = 80
NM = EPSUB // M
G = M // LANES
NP = 10240
HH = 128


def _proj_body(x_ref, w_ref, b_ref, k_ref, v_ref, q_ref, s_ref):
    acc = jnp.dot(x_ref[...], w_ref[...], preferred_element_type=jnp.float32)
    for c in range(2):
        k_ref[c] = acc[:, 128 * c:128 * c + 128] + b_ref[c][None, :]
        v_ref[c] = acc[:, 512 + 128 * c:512 + 128 * c + 128] \
            + b_ref[4 + c][None, :]
        q_ref[c] = acc[:, 256 + 128 * c:256 + 128 * c + 128] \
            + b_ref[2 + c][None, :]
        s_ref[c] = acc[:, 768 + 128 * c:768 + 128 * c + 128] \
            + b_ref[6 + c][None, :]


def _proj(x, wcat, bcat):
    return pl.pallas_call(
        _proj_body,
        out_shape=(
            jax.ShapeDtypeStruct((2, N, 128), jnp.float32),
            jax.ShapeDtypeStruct((2, N, 128), jnp.float32),
            jax.ShapeDtypeStruct((2, N, 128), jnp.float32),
            jax.ShapeDtypeStruct((2, N, 128), jnp.float32),
        ),
    )(x, wcat, bcat)


def _bnrelu_body(agg_ref, s_ref, g_ref, b_ref, out_ref):
    for c in range(2):
        t = agg_ref[c] + s_ref[c]
        m = jnp.mean(t, axis=0)
        t0 = t - m[None, :]
        v = jnp.mean(t0 * t0, axis=0)
        y = g_ref[c][None, :] * t0 * lax.rsqrt(v + EPS) + b_ref[c][None, :]
        out_ref[:, 128 * c:128 * (c + 1)] = jnp.maximum(y, 0.0)


def _bnrelu(agg, s, g, b):
    return pl.pallas_call(
        _bnrelu_body,
        out_shape=jax.ShapeDtypeStruct((N, HC), jnp.float32),
    )(agg, s, g.reshape(2, 128), b.reshape(2, 128))


def _final_body(h_ref, w_ref, b_ref, g_ref, bn_ref, out_ref):
    t = jnp.dot(h_ref[...], w_ref[...], preferred_element_type=jnp.float32)
    t = t + b_ref[...].reshape(1, OUT)
    m = jnp.mean(t, axis=0)
    t0 = t - m[None, :]
    v = jnp.mean(t0 * t0, axis=0)
    y = g_ref[...].reshape(1, OUT) * t0 * lax.rsqrt(v + EPS) \
        + bn_ref[...].reshape(1, OUT)
    out_ref[...] = jnp.maximum(y, 0.0)


def _final(h, w, b, g, bn):
    return pl.pallas_call(
        _final_body,
        out_shape=jax.ShapeDtypeStruct((N, OUT), jnp.float32),
    )(h, w, b.reshape(2, 128), g.reshape(2, 128), bn.reshape(2, 128))


def _attn_body(k2, v2, q2, srch, dsth, out,
               srcC, dstC, gs, gd, gv, sdst, sdst1,
               kb, vb, qb, mbuf,
               exb0, exb1, tr0, tr1, zrow, dfinal, agg,
               semA, semB, semC):
    c = lax.axis_index("c")
    s = lax.axis_index("s")
    cn = c * N
    cnv = jnp.full((LANES,), cn, jnp.int32)
    npv = jnp.full((LANES,), NP, jnp.int32)
    ziv = jnp.zeros((LANES,), jnp.int32)
    iot = lax.iota(jnp.int32, LANES)
    ib16 = iot * LANES
    zf = jnp.zeros((LANES,), jnp.float32)

    def zrow_body(i, _):
        zrow[pl.ds(i * LANES, LANES)] = zf
        return 0
    lax.fori_loop(0, 1280 // LANES, zrow_body, 0)

    def zbuf_body(r, _):
        for j in range(8):
            mbuf[r, pl.ds(j * LANES, LANES)] = zf
        return 0
    lax.fori_loop(0, M, zbuf_body, 0)

    for t in range(8):
        ch = s + NS * t

        @pl.when(ch < NM)
        def _():
            pltpu.sync_copy(mbuf, agg.at[pl.ds(ch * M, M)])
    pltpu.sync_copy(zrow, dfinal.at[pl.ds(s * 1280, 1280)])
    plsc.subcore_barrier()

    ebase = s * EPSUB

    def build(idx_ref, from_ref, addv):
        def body(g, _):
            idx_ref[pl.ds(g * LANES, LANES)] = \
                from_ref[pl.ds(g * LANES, LANES)] + addv
            return 0
        lax.fori_loop(0, G, body, 0)

    def compute_dots():
        def group(g, _):
            def edge(e16, _):
                e = g * LANES + e16
                acc0 = zf
                acc1 = zf
                for j in range(8):
                    kvv = kb[e, pl.ds(j * LANES, LANES)]
                    qv = qb[e, pl.ds(j * LANES, LANES)]
                    if j < 4:
                        acc0 = acc0 + kvv * qv
                    else:
                        acc1 = acc1 + kvv * qv
                tr0[pl.ds(e16 * LANES, LANES)] = acc0
                tr1[pl.ds(e16 * LANES, LANES)] = acc1
                return 0
            lax.fori_loop(0, LANES, edge, 0)

            s0 = zf
            s1 = zf
            for j in range(LANES):
                s0 = s0 + plsc.load_gather(tr0, [ib16 + j])
                s1 = s1 + plsc.load_gather(tr1, [ib16 + j])
            exb0[pl.ds(g * LANES, LANES)] = jnp.exp(s0 * 0.125)
            exb1[pl.ds(g * LANES, LANES)] = jnp.exp(s1 * 0.125)
            return 0
        lax.fori_loop(0, G, group, 0)

    def scale_and_scatter():
        def edge(e, _):
            ev = jnp.full((LANES,), e, jnp.int32)
            b0 = plsc.load_gather(exb0, [ev])
            b1 = plsc.load_gather(exb1, [ev])
            for j in range(8):
                bb = b0 if j < 4 else b1
                mbuf[e, pl.ds(j * LANES, LANES)] = \
                    vb[e, pl.ds(j * LANES, LANES)] * bb
            return 0
        lax.fori_loop(0, M, edge, 0)

        pltpu.sync_copy(exb0, dfinal.at[sdst], add=True)
        pltpu.sync_copy(exb1, dfinal.at[sdst1], add=True)
        pltpu.sync_copy(mbuf, agg.at[sdst], add=True)

    def fire_idx(i):
        pltpu.async_copy(srch.at[pl.ds(ebase + i * M, M)], srcC, semC)
        pltpu.async_copy(dsth.at[pl.ds(ebase + i * M, M)], dstC, semC)

    def wait_idx(i):
        pltpu.make_async_copy(srch.at[pl.ds(ebase + i * M, M)],
                              srcC, semC).wait()
        pltpu.make_async_copy(dsth.at[pl.ds(ebase + i * M, M)],
                              dstC, semC).wait()

    fire_idx(0)
    wait_idx(0)
    build(gs, srcC, cnv)
    build(gd, dstC, cnv)
    pltpu.async_copy(k2.at[gs], kb, semA)
    pltpu.async_copy(q2.at[gd], qb, semA)

    def chunk(i, _):
        build(sdst, dstC, ziv)
        build(sdst1, dstC, npv)
        build(gv, srcC, cnv)

        @pl.when(i + 1 < NM)
        def _():
            fire_idx(i + 1)

        pltpu.make_async_copy(k2.at[gs], kb, semA).wait()
        pltpu.make_async_copy(q2.at[gd], qb, semA).wait()
        pltpu.async_copy(v2.at[gv], vb, semB)
        compute_dots()

        @pl.when(i + 1 < NM)
        def _():
            wait_idx(i + 1)
            build(gs, srcC, cnv)
            build(gd, dstC, cnv)
            pltpu.async_copy(k2.at[gs], kb, semA)
            pltpu.async_copy(q2.at[gd], qb, semA)

        pltpu.make_async_copy(v2.at[gv], vb, semB).wait()
        scale_and_scatter()
        return 0
    lax.fori_loop(0, NM, chunk, 0)

    plsc.subcore_barrier()

    for t in range(8):
        ch = s + NS * t

        @pl.when(ch < NM)
        def _():
            base = ch * M
            pltpu.sync_copy(agg.at[pl.ds(base, M)], mbuf)
            pltpu.sync_copy(dfinal.at[pl.ds(base, M)], exb0)
            pltpu.sync_copy(dfinal.at[pl.ds(NP + base, M)], exb1)

            def recip(g, _):
                exb0[pl.ds(g * LANES, LANES)] = \
                    1.0 / (exb0[pl.ds(g * LANES, LANES)] + 1e-16)
                exb1[pl.ds(g * LANES, LANES)] = \
                    1.0 / (exb1[pl.ds(g * LANES, LANES)] + 1e-16)
                return 0
            lax.fori_loop(0, G, recip, 0)

            def row(r, _):
                rv = jnp.full((LANES,), r, jnp.int32)
                b0 = plsc.load_gather(exb0, [rv])
                b1 = plsc.load_gather(exb1, [rv])
                for j in range(8):
                    bb = b0 if j < 4 else b1
                    mbuf[r, pl.ds(j * LANES, LANES)] = \
                        mbuf[r, pl.ds(j * LANES, LANES)] * bb
                return 0
            lax.fori_loop(0, M, row, 0)

            pltpu.sync_copy(mbuf, out.at[c].at[pl.ds(base, M)])


@functools.partial(
    pl.kernel,
    out_type=jax.ShapeDtypeStruct((2, N, HH), jnp.float32),
    mesh=plsc.VectorSubcoreMesh(core_axis_name="c", subcore_axis_name="s"),
    compiler_params=pltpu.CompilerParams(needs_layout_passes=False),
    scratch_types=[
        pltpu.VMEM((M,), jnp.int32),
        pltpu.VMEM((M,), jnp.int32),
        pltpu.VMEM((M,), jnp.int32),
        pltpu.VMEM((M,), jnp.int32),
        pltpu.VMEM((M,), jnp.int32),
        pltpu.VMEM((M,), jnp.int32),
        pltpu.VMEM((M,), jnp.int32),
        pltpu.VMEM((M, HH), jnp.float32),
        pltpu.VMEM((M, HH), jnp.float32),
        pltpu.VMEM((M, HH), jnp.float32),
        pltpu.VMEM((M, HH), jnp.float32),
        pltpu.VMEM((M,), jnp.float32),
        pltpu.VMEM((M,), jnp.float32),
        pltpu.VMEM((LANES * LANES,), jnp.float32),
        pltpu.VMEM((LANES * LANES,), jnp.float32),
        pltpu.VMEM((1280,), jnp.float32),
        pltpu.VMEM_SHARED((2 * NP,), jnp.float32),
        pltpu.VMEM_SHARED((N, HH), jnp.float32),
        pltpu.SemaphoreType.DMA,
        pltpu.SemaphoreType.DMA,
        pltpu.SemaphoreType.DMA,
    ],
)
def _attn(k2, v2, q2, srch, dsth, out, *scratch):
    _attn_body(k2, v2, q2, srch, dsth, out, *scratch)


def _tconv(x, src, dst, wcat, bcat):
    k, v, q, s2 = _proj(x, wcat, bcat)
    k2 = k.reshape(2 * N, HH)
    v2 = v.reshape(2 * N, HH)
    q2 = q.reshape(2 * N, HH)
    agg = _attn(k2, v2, q2, src, dst)
    return agg, s2


def kernel(x, edge_index, l0_Wk, l0_bk, l0_Wq, l0_bq, l0_Wv, l0_bv, l0_Ws,
           l0_bs, l1_Wk, l1_bk, l1_Wq, l1_bq, l1_Wv, l1_bv, l1_Ws, l1_bs,
           bn0_g, bn0_b, bn1_g, bn1_b, bn2_g, bn2_b, Wout, bout):
    src = edge_index[0]
    dst = edge_index[1]
    w0 = jnp.concatenate([l0_Wk, l0_Wq, l0_Wv, l0_Ws], axis=1)
    b0 = jnp.concatenate([l0_bk, l0_bq, l0_bv, l0_bs]).reshape(8, 128)
    w1 = jnp.concatenate([l1_Wk, l1_Wq, l1_Wv, l1_Ws], axis=1)
    b1 = jnp.concatenate([l1_bk, l1_bq, l1_bv, l1_bs]).reshape(8, 128)

    agg, s2 = _tconv(x, src, dst, w0, b0)
    h = _bnrelu(agg, s2, bn0_g, bn0_b)
    agg, s2 = _tconv(h, src, dst, w1, b1)
    h = _bnrelu(agg, s2, bn1_g, bn1_b)
    return _final(h, Wout, bout, bn2_g, bn2_b)
